# baseline (speedup 1.0000x reference)
import functools

import jax
import jax.numpy as jnp
from jax import lax
from jax.experimental import pallas as pl
from jax.experimental.pallas import tpu as pltpu
from jax.experimental.pallas import tpu_sc as plsc

B, L, C = 4096, 50, 16
D = 128
LP = 64
NC, NS = 2, 16
NW = NC * NS
BPW = B // NW
WCH = 2
WIDX = WCH * LP
NWCH = BPW // WCH
CCH = 128
NCCH = (BPW * C) // CCH
DSL = D // 16
G = 4


def _sc_mesh():
    return plsc.VectorSubcoreMesh(core_axis_name="c", subcore_axis_name="s")


@functools.partial(
    pl.kernel,
    mesh=_sc_mesh(),
    out_type=[
        jax.ShapeDtypeStruct((B, D), jnp.float32),
        jax.ShapeDtypeStruct((B * C, D), jnp.float32),
    ],
    scratch_types=[
        pltpu.VMEM((BPW * LP,), jnp.int32),
        pltpu.VMEM((G, CCH, D), jnp.float32),
        pltpu.VMEM((BPW, D), jnp.float32),
        pltpu.VMEM((BPW * C,), jnp.int32),
        pltpu.SemaphoreType.DMA,
        pltpu.SemaphoreType.DMA,
        pltpu.SemaphoreType.DMA,
        pltpu.SemaphoreType.DMA,
    ],
)
def _sc_gather(wemb, eemb, wids, cids, psum_out, cands_out,
               widx_v, rows_v, acc_v, cidx_v, s0, s1, s2, s3):
    sems = (s0, s1, s2, s3)
    wid = lax.axis_index("s") * NC + lax.axis_index("c")
    wbase = wid * BPW

    pltpu.sync_copy(wids.at[pl.ds(wbase * LP, BPW * LP)], widx_v)
    pltpu.sync_copy(cids.at[pl.ds(wbase * C, BPW * C)], cidx_v)

    def wdesc(k, g):
        off = pl.multiple_of(k * WIDX, 128)
        return pltpu.make_async_copy(
            wemb.at[widx_v.at[pl.ds(off, WIDX)]],
            rows_v.at[g, pl.ds(0, WIDX)], sems[g])

    for g in range(G):
        wdesc(g, g).start()

    def wgroup(m, carry):
        for g in range(G):
            k = m * G + g
            wdesc(k, g).wait()
            for rr in range(WCH):
                def add_rows(j, accs, _rr=rr, _g=g):
                    base = _rr * LP + j * 2
                    t = tuple(
                        accs[d] + rows_v[_g, base, pl.ds(d * 16, 16)]
                        for d in range(DSL))
                    return tuple(
                        t[d] + rows_v[_g, base + 1, pl.ds(d * 16, 16)]
                        for d in range(DSL))
                accs = lax.fori_loop(
                    0, LP // 2, add_rows,
                    tuple(jnp.zeros((16,), jnp.float32) for _ in range(DSL)))
                r = k * WCH + rr
                for d in range(DSL):
                    acc_v[r, pl.ds(d * 16, 16)] = accs[d]

            @pl.when(k + G < NWCH)
            def _fire_next(_k=k, _g=g):
                wdesc(_k + G, _g).start()
        return carry

    lax.fori_loop(0, NWCH // G, wgroup, 0)
    pltpu.sync_copy(acc_v, psum_out.at[pl.ds(wbase, BPW)])

    def cdesc(k, g):
        off = pl.multiple_of(k * CCH, 8)
        return pltpu.make_async_copy(
            eemb.at[cidx_v.at[pl.ds(off, CCH)]], rows_v.at[g], sems[g])

    for g in range(G):
        cdesc(g, g).start()

    def cgroup(m, carry):
        for g in range(G):
            k = m * G + g
            cdesc(k, g).wait()
            dst = pl.multiple_of(wbase * C + k * CCH, 8)
            pltpu.sync_copy(rows_v.at[g], cands_out.at[pl.ds(dst, CCH)])

            @pl.when(k + G < NCCH)
            def _fire_next(_k=k, _g=g):
                cdesc(_k + G, _g).start()
        return carry

    lax.fori_loop(0, NCCH // G, cgroup, 0)


TB = 256


def _tc_body(ids_ref, psum_ref, w_ref, b_ref, cands_ref, out_ref):
    ids = ids_ref[...]
    cnt = jnp.sum((ids != 0).astype(jnp.float32), axis=1, keepdims=True)
    pooled = psum_ref[...] / jnp.maximum(cnt, 1.0)
    proj = lax.dot_general(
        pooled, w_ref[...], (((1,), (1,)), ((), ())),
        preferred_element_type=jnp.float32,
    ) + b_ref[...]
    cands = cands_ref[...].reshape(TB, C, D)
    out_ref[...] = jnp.sum(cands * proj[:, None, :], axis=-1)


def kernel(word_emb, ent_emb, W, b, word_ids, cand_ent_ids):
    wids = word_ids.astype(jnp.int32)
    cids = cand_ent_ids.astype(jnp.int32)
    wids_pad = jnp.pad(wids, ((0, 0), (0, LP - L))).reshape(-1)
    cids_flat = cids.reshape(-1)

    psum, cands = _sc_gather(word_emb, ent_emb, wids_pad, cids_flat)

    scores = pl.pallas_call(
        _tc_body,
        grid=(B // TB,),
        in_specs=[
            pl.BlockSpec((TB, L), lambda i: (i, 0)),
            pl.BlockSpec((TB, D), lambda i: (i, 0)),
            pl.BlockSpec((D, D), lambda i: (0, 0)),
            pl.BlockSpec((1, D), lambda i: (0, 0)),
            pl.BlockSpec((TB * C, D), lambda i: (i, 0)),
        ],
        out_specs=pl.BlockSpec((TB, C), lambda i: (i, 0)),
        out_shape=jax.ShapeDtypeStruct((B, C), jnp.float32),
    )(wids, psum, W, b.reshape(1, D), cands)
    return scores


# speedup vs baseline: 19.9757x; 19.9757x over previous
import functools

import jax
import jax.numpy as jnp
from jax import lax
from jax.experimental import pallas as pl
from jax.experimental.pallas import tpu as pltpu
from jax.experimental.pallas import tpu_sc as plsc

B, L, C = 4096, 50, 16
D = 128
LP = 56
NC, NS = 2, 16
NW = NC * NS
BPW = B // NW
WCH = 2
WIDX = WCH * LP
NWCH = BPW // WCH
CCH = 128
NCCH = (BPW * C) // CCH
DSL = D // 16
G = 4


def _sc_mesh():
    return plsc.VectorSubcoreMesh(core_axis_name="c", subcore_axis_name="s")


@functools.partial(
    pl.kernel,
    mesh=_sc_mesh(),
    out_type=[
        jax.ShapeDtypeStruct((B, D), jnp.float32),
        jax.ShapeDtypeStruct((B * C, D), jnp.float32),
    ],
    scratch_types=[
        pltpu.VMEM((BPW * LP,), jnp.int32),
        pltpu.VMEM((G, CCH, D), jnp.float32),
        pltpu.VMEM((BPW, D), jnp.float32),
        pltpu.VMEM((BPW * C,), jnp.int32),
        pltpu.SemaphoreType.DMA,
        pltpu.SemaphoreType.DMA,
        pltpu.SemaphoreType.DMA,
        pltpu.SemaphoreType.DMA,
    ],
)
def _sc_gather(wemb, eemb, wids, cids, psum_out, cands_out,
               widx_v, rows_v, acc_v, cidx_v, s0, s1, s2, s3):
    sems = (s0, s1, s2, s3)
    wid = lax.axis_index("s") * NC + lax.axis_index("c")
    wbase = wid * BPW

    pltpu.sync_copy(wids.at[pl.ds(wbase * LP, BPW * LP)], widx_v)
    pltpu.sync_copy(cids.at[pl.ds(wbase * C, BPW * C)], cidx_v)

    def wdesc(k, g):
        off = pl.multiple_of(k * WIDX, 8)
        return pltpu.make_async_copy(
            wemb.at[widx_v.at[pl.ds(off, WIDX)]],
            rows_v.at[g, pl.ds(0, WIDX)], sems[g])

    for g in range(G):
        wdesc(g, g).start()

    def wgroup(m, carry):
        for g in range(G):
            k = m * G + g
            wdesc(k, g).wait()
            for rr in range(WCH):
                def add_rows(j, accs, _rr=rr, _g=g):
                    base = _rr * LP + j * 2
                    t = tuple(
                        accs[d] + rows_v[_g, base, pl.ds(d * 16, 16)]
                        for d in range(DSL))
                    return tuple(
                        t[d] + rows_v[_g, base + 1, pl.ds(d * 16, 16)]
                        for d in range(DSL))
                accs = lax.fori_loop(
                    0, L // 2, add_rows,
                    tuple(jnp.zeros((16,), jnp.float32) for _ in range(DSL)))
                r = k * WCH + rr
                for d in range(DSL):
                    acc_v[r, pl.ds(d * 16, 16)] = accs[d]

            @pl.when(k + G < NWCH)
            def _fire_next(_k=k, _g=g):
                wdesc(_k + G, _g).start()
        return carry

    lax.fori_loop(0, NWCH // G, wgroup, 0)
    pltpu.sync_copy(acc_v, psum_out.at[pl.ds(wbase, BPW)])

    def cdesc(k, g):
        off = pl.multiple_of(k * CCH, 8)
        return pltpu.make_async_copy(
            eemb.at[cidx_v.at[pl.ds(off, CCH)]], rows_v.at[g], sems[g])

    for g in range(G):
        cdesc(g, g).start()

    def cgroup(m, carry):
        for g in range(G):
            k = m * G + g
            cdesc(k, g).wait()
            dst = pl.multiple_of(wbase * C + k * CCH, 8)
            pltpu.sync_copy(rows_v.at[g], cands_out.at[pl.ds(dst, CCH)])

            @pl.when(k + G < NCCH)
            def _fire_next(_k=k, _g=g):
                cdesc(_k + G, _g).start()
        return carry

    lax.fori_loop(0, NCCH // G, cgroup, 0)


TB = 256


def _tc_body(ids_ref, psum_ref, w_ref, b_ref, cands_ref, out_ref):
    ids = ids_ref[...]
    cnt = jnp.sum((ids != 0).astype(jnp.float32), axis=1, keepdims=True)
    pooled = psum_ref[...] / jnp.maximum(cnt, 1.0)
    proj = lax.dot_general(
        pooled, w_ref[...], (((1,), (1,)), ((), ())),
        preferred_element_type=jnp.float32,
    ) + b_ref[...]
    cands = cands_ref[...].reshape(TB, C, D)
    out_ref[...] = jnp.sum(cands * proj[:, None, :], axis=-1)


def kernel(word_emb, ent_emb, W, b, word_ids, cand_ent_ids):
    wids = word_ids.astype(jnp.int32)
    cids = cand_ent_ids.astype(jnp.int32)
    nrows = jnp.int32(word_emb.shape[0])
    spread = ((jnp.arange(B * LP, dtype=jnp.int32) * 7919) % nrows).reshape(B, LP)
    col = jnp.arange(LP, dtype=jnp.int32)[None, :]
    wids_pad = jnp.where(
        col < L, jnp.pad(wids, ((0, 0), (0, LP - L))), spread
    ).reshape(-1)
    cids_flat = cids.reshape(-1)

    psum, cands = _sc_gather(word_emb, ent_emb, wids_pad, cids_flat)

    scores = pl.pallas_call(
        _tc_body,
        grid=(B // TB,),
        in_specs=[
            pl.BlockSpec((TB, L), lambda i: (i, 0)),
            pl.BlockSpec((TB, D), lambda i: (i, 0)),
            pl.BlockSpec((D, D), lambda i: (0, 0)),
            pl.BlockSpec((1, D), lambda i: (0, 0)),
            pl.BlockSpec((TB * C, D), lambda i: (i, 0)),
        ],
        out_specs=pl.BlockSpec((TB, C), lambda i: (i, 0)),
        out_shape=jax.ShapeDtypeStruct((B, C), jnp.float32),
    )(wids, psum, W, b.reshape(1, D), cands)
    return scores


# speedup vs baseline: 20.3259x; 1.0175x over previous
import functools

import jax
import jax.numpy as jnp
from jax import lax
from jax.experimental import pallas as pl
from jax.experimental.pallas import tpu as pltpu
from jax.experimental.pallas import tpu_sc as plsc

B, L, C = 4096, 50, 16
D = 128
LP = 56
NC, NS = 2, 16
NW = NC * NS
BPW = B // NW
WCH = 1
WIDX = WCH * LP
NWCH = BPW // WCH
CCH = 64
NCCH = (BPW * C) // CCH
DSL = D // 16
G = 8


def _sc_mesh():
    return plsc.VectorSubcoreMesh(core_axis_name="c", subcore_axis_name="s")


@functools.partial(
    pl.kernel,
    mesh=_sc_mesh(),
    out_type=[
        jax.ShapeDtypeStruct((B, D), jnp.float32),
        jax.ShapeDtypeStruct((B * C, D), jnp.float32),
    ],
    scratch_types=[
        pltpu.VMEM((BPW * LP,), jnp.int32),
        pltpu.VMEM((G, CCH, D), jnp.float32),
        pltpu.VMEM((BPW, D), jnp.float32),
        pltpu.VMEM((BPW * C,), jnp.int32),
        pltpu.SemaphoreType.DMA,
        pltpu.SemaphoreType.DMA,
        pltpu.SemaphoreType.DMA,
        pltpu.SemaphoreType.DMA,
        pltpu.SemaphoreType.DMA,
        pltpu.SemaphoreType.DMA,
        pltpu.SemaphoreType.DMA,
        pltpu.SemaphoreType.DMA,
    ],
)
def _sc_gather(wemb, eemb, wids, cids, psum_out, cands_out,
               widx_v, rows_v, acc_v, cidx_v,
               s0, s1, s2, s3, s4, s5, s6, s7):
    sems = (s0, s1, s2, s3, s4, s5, s6, s7)
    wid = lax.axis_index("s") * NC + lax.axis_index("c")
    wbase = wid * BPW

    pltpu.sync_copy(wids.at[pl.ds(wbase * LP, BPW * LP)], widx_v)
    pltpu.sync_copy(cids.at[pl.ds(wbase * C, BPW * C)], cidx_v)

    def wdesc(k, g):
        off = pl.multiple_of(k * WIDX, 8)
        return pltpu.make_async_copy(
            wemb.at[widx_v.at[pl.ds(off, WIDX)]],
            rows_v.at[g, pl.ds(0, WIDX)], sems[g])

    for g in range(G):
        wdesc(g, g).start()

    def wgroup(m, carry):
        for g in range(G):
            k = m * G + g
            wdesc(k, g).wait()
            for rr in range(WCH):
                def add_rows(j, accs, _rr=rr, _g=g):
                    base = _rr * LP + j * 2
                    t = tuple(
                        accs[d] + rows_v[_g, base, pl.ds(d * 16, 16)]
                        for d in range(DSL))
                    return tuple(
                        t[d] + rows_v[_g, base + 1, pl.ds(d * 16, 16)]
                        for d in range(DSL))
                accs = lax.fori_loop(
                    0, L // 2, add_rows,
                    tuple(jnp.zeros((16,), jnp.float32) for _ in range(DSL)))
                r = k * WCH + rr
                for d in range(DSL):
                    acc_v[r, pl.ds(d * 16, 16)] = accs[d]

            @pl.when(k + G < NWCH)
            def _fire_next(_k=k, _g=g):
                wdesc(_k + G, _g).start()
        return carry

    lax.fori_loop(0, NWCH // G, wgroup, 0)
    pltpu.sync_copy(acc_v, psum_out.at[pl.ds(wbase, BPW)])

    def cdesc(k, g):
        off = pl.multiple_of(k * CCH, 8)
        return pltpu.make_async_copy(
            eemb.at[cidx_v.at[pl.ds(off, CCH)]], rows_v.at[g], sems[g])

    for g in range(G):
        cdesc(g, g).start()

    def cgroup(m, carry):
        for g in range(G):
            k = m * G + g
            cdesc(k, g).wait()
            dst = pl.multiple_of(wbase * C + k * CCH, 8)
            pltpu.sync_copy(rows_v.at[g], cands_out.at[pl.ds(dst, CCH)])

            @pl.when(k + G < NCCH)
            def _fire_next(_k=k, _g=g):
                cdesc(_k + G, _g).start()
        return carry

    lax.fori_loop(0, NCCH // G, cgroup, 0)


TB = 256


def _tc_body(ids_ref, psum_ref, w_ref, b_ref, cands_ref, out_ref):
    ids = ids_ref[...]
    cnt = jnp.sum((ids != 0).astype(jnp.float32), axis=1, keepdims=True)
    pooled = psum_ref[...] / jnp.maximum(cnt, 1.0)
    proj = lax.dot_general(
        pooled, w_ref[...], (((1,), (1,)), ((), ())),
        preferred_element_type=jnp.float32,
    ) + b_ref[...]
    cands = cands_ref[...].reshape(TB, C, D)
    out_ref[...] = jnp.sum(cands * proj[:, None, :], axis=-1)


def kernel(word_emb, ent_emb, W, b, word_ids, cand_ent_ids):
    wids = word_ids.astype(jnp.int32)
    cids = cand_ent_ids.astype(jnp.int32)
    nrows = jnp.int32(word_emb.shape[0])
    spread = ((jnp.arange(B * LP, dtype=jnp.int32) * 7919) % nrows).reshape(B, LP)
    col = jnp.arange(LP, dtype=jnp.int32)[None, :]
    wids_pad = jnp.where(
        col < L, jnp.pad(wids, ((0, 0), (0, LP - L))), spread
    ).reshape(-1)
    cids_flat = cids.reshape(-1)

    psum, cands = _sc_gather(word_emb, ent_emb, wids_pad, cids_flat)

    scores = pl.pallas_call(
        _tc_body,
        grid=(B // TB,),
        in_specs=[
            pl.BlockSpec((TB, L), lambda i: (i, 0)),
            pl.BlockSpec((TB, D), lambda i: (i, 0)),
            pl.BlockSpec((D, D), lambda i: (0, 0)),
            pl.BlockSpec((1, D), lambda i: (0, 0)),
            pl.BlockSpec((TB * C, D), lambda i: (i, 0)),
        ],
        out_specs=pl.BlockSpec((TB, C), lambda i: (i, 0)),
        out_shape=jax.ShapeDtypeStruct((B, C), jnp.float32),
    )(wids, psum, W, b.reshape(1, D), cands)
    return scores
